# baseline (speedup 1.0000x reference)
import jax
import jax.numpy as jnp
from jax import lax
from jax.experimental import pallas as pl
from jax.experimental.pallas import tpu as pltpu

_NBUF = 4


def _unpool_body(v_hbm, x_hbm, o_hbm, xbuf, vbuf, lsem_x, lsem_v, ssem):
    i = pl.program_id(0)
    n = pl.num_programs(0)
    slot = lax.rem(i, _NBUF)
    nslot = lax.rem(i + 1, _NBUF)

    def load(b, s):
        pltpu.make_async_copy(x_hbm.at[b], xbuf.at[s], lsem_x.at[s]).start()
        pltpu.make_async_copy(v_hbm.at[b], vbuf.at[s], lsem_v.at[s]).start()

    @pl.when(i == 0)
    def _():
        load(0, 0)

    @pl.when(jnp.logical_and(i + 1 < n, i + 1 >= _NBUF))
    def _():
        pltpu.make_async_copy(xbuf.at[nslot], o_hbm.at[i + 1 - _NBUF],
                              ssem.at[nslot]).wait()

    @pl.when(i + 1 < n)
    def _():
        load(i + 1, nslot)

    pltpu.make_async_copy(x_hbm.at[i], xbuf.at[slot], lsem_x.at[slot]).wait()
    pltpu.make_async_copy(v_hbm.at[i], vbuf.at[slot], lsem_v.at[slot]).wait()

    x = xbuf[slot]
    v2 = vbuf[slot]
    h, w2, c2 = x.shape

    wx = jnp.maximum(x, pltpu.roll(x, c2 // 2, axis=2))
    wr = wx.reshape(h // 2, 2, w2, c2)
    m = jnp.maximum(wr[:, 0], wr[:, 1])

    xr = x.reshape(h // 2, 2, w2, c2)
    oe = jnp.where(xr[:, 0] == m, v2, 0.0)
    oo = jnp.where(xr[:, 1] == m, v2, 0.0)
    xbuf[slot] = jnp.stack([oe, oo], axis=1).reshape(h, w2, c2)

    pltpu.make_async_copy(xbuf.at[slot], o_hbm.at[i], ssem.at[slot]).start()

    @pl.when(i == n - 1)
    def _():
        for s in range(_NBUF):
            pltpu.make_async_copy(xbuf.at[s], o_hbm.at[0], ssem.at[s]).wait()


def kernel(input_tensor, pool_input):
    B, H, W, C = pool_input.shape
    Ho, Wo = H // 2, W // 2
    x2 = pool_input.reshape(B, H, Wo, 2 * C)
    v2 = jnp.concatenate([input_tensor, input_tensor], axis=-1)

    out = pl.pallas_call(
        _unpool_body,
        grid=(B,),
        in_specs=[
            pl.BlockSpec(memory_space=pl.ANY),
            pl.BlockSpec(memory_space=pl.ANY),
        ],
        out_specs=pl.BlockSpec(memory_space=pl.ANY),
        out_shape=jax.ShapeDtypeStruct((B, H, Wo, 2 * C), pool_input.dtype),
        scratch_shapes=[
            pltpu.VMEM((_NBUF, H, Wo, 2 * C), pool_input.dtype),
            pltpu.VMEM((_NBUF, Ho, Wo, 2 * C), pool_input.dtype),
            pltpu.SemaphoreType.DMA((_NBUF,)),
            pltpu.SemaphoreType.DMA((_NBUF,)),
            pltpu.SemaphoreType.DMA((_NBUF,)),
        ],
        compiler_params=pltpu.CompilerParams(
            dimension_semantics=("arbitrary",),
        ),
    )(v2, x2)
    return out.reshape(B, H, W, C)
